# Initial kernel scaffold; baseline (speedup 1.0000x reference)
import jax
import jax.numpy as jnp
from jax.experimental import pallas as pl


def kernel(x, A_physical, W, b, alpha):
    raise NotImplementedError("write your pallas kernel here")


# speedup vs baseline: 11.4485x; 11.4485x over previous
import jax
import jax.numpy as jnp
from jax.experimental import pallas as pl

_K = 10
_ROWS = 256
_H = 16


def _cumsum_lanes(x):
    n = x.shape[-1]
    shift = 1
    while shift < n:
        shifted = jnp.concatenate(
            [jnp.zeros(x.shape[:-1] + (shift,), x.dtype), x[..., :-shift]], axis=-1)
        x = x + shifted
        shift *= 2
    return x


def _tc_kernel(embt_ref, emb_rows_ref, alpha_ref, phys_ref, out_ref):
    embt = embt_ref[0]
    emb_rows = emb_rows_ref[0]
    c = jax.nn.sigmoid(alpha_ref[0, 0])

    a = jax.lax.dot_general(emb_rows, embt, (((1,), (0,)), ((), ())),
                            preferred_element_type=jnp.float32)
    a = jnp.maximum(a, 0.0)

    r = a.shape[0]
    cur = jnp.full((r, 1), jnp.inf, dtype=jnp.float32)
    cnt = jnp.zeros((r, 1), dtype=jnp.int32)
    thr = jnp.zeros((r, 1), dtype=jnp.float32)
    row_max = jnp.zeros((r, 1), dtype=jnp.float32)
    for k in range(_K):
        masked = jnp.where(a < cur, a, -1.0)
        d = jnp.max(masked, axis=1, keepdims=True)
        dcnt = jnp.sum((a == d).astype(jnp.int32), axis=1, keepdims=True)
        take = cnt < _K
        thr = jnp.where(take, d, thr)
        cnt = cnt + jnp.where(take, dcnt, 0)
        if k == 0:
            row_max = d
        cur = d

    mask_gt = a > thr
    cnt_gt = jnp.sum(mask_gt.astype(jnp.int32), axis=1, keepdims=True)
    extra = _K - cnt_gt
    mask_eq = a == thr
    rank = _cumsum_lanes(mask_eq.astype(jnp.int32))
    sel = jnp.logical_or(mask_gt, jnp.logical_and(mask_eq, rank <= extra))

    e = jnp.exp(a - row_max)
    z = jnp.sum(jnp.where(sel, e, 0.0), axis=1, keepdims=True)

    phys = phys_ref[...]
    psum = jnp.sum(phys, axis=1, keepdims=True) + 1e-8
    out = (c / psum) * phys + jnp.where(sel, ((1.0 - c) / z) * e, 0.0)
    out_ref[0, :, :] = out


def kernel(x, A_physical, W, b, alpha):
    bsz, _, n, _ = x.shape
    state = x[:, -1, :, :]
    emb = jnp.tanh(state @ W + b)
    embt = jnp.swapaxes(emb, 1, 2)
    alpha2 = jnp.asarray(alpha, jnp.float32).reshape(1, 1)
    grid = (n // _ROWS, bsz)
    return pl.pallas_call(
        _tc_kernel,
        grid=grid,
        in_specs=[
            pl.BlockSpec((1, _H, n), lambda i, bb: (bb, 0, 0)),
            pl.BlockSpec((1, _ROWS, _H), lambda i, bb: (bb, i, 0)),
            pl.BlockSpec((1, 1), lambda i, bb: (0, 0)),
            pl.BlockSpec((_ROWS, n), lambda i, bb: (i, 0)),
        ],
        out_specs=pl.BlockSpec((1, _ROWS, n), lambda i, bb: (bb, i, 0)),
        out_shape=jax.ShapeDtypeStruct((bsz, n, n), jnp.float32),
    )(embt, emb, alpha2, A_physical)
